# Initial kernel scaffold; baseline (speedup 1.0000x reference)
import jax
import jax.numpy as jnp
from jax.experimental import pallas as pl


def kernel(x, edge_index, W_neigh, b_neigh, W_self, b_self):
    raise NotImplementedError("write your pallas kernel here")


# speedup vs baseline: 8.0213x; 8.0213x over previous
import functools

import jax
import jax.numpy as jnp
from jax import lax
from jax.experimental import pallas as pl
from jax.experimental.pallas import tpu as pltpu
from jax.experimental.pallas import tpu_sc as plsc

_NC = 2
_NS = 16
_C = 80
_ZR = 16


def _neighbor_partials(eidx, x, npad):
    n, d = npad, x.shape[1]
    nch = eidx.shape[2]
    rpt = n // _NS

    mesh = plsc.VectorSubcoreMesh(core_axis_name="c", subcore_axis_name="s")

    @functools.partial(
        pl.kernel,
        out_type=jax.ShapeDtypeStruct((_NC, n, d), jnp.float32),
        mesh=mesh,
        scratch_types=[
            pltpu.VMEM((nch, _C), jnp.int32),
            pltpu.VMEM((nch, _C), jnp.int32),
            pltpu.VMEM((_C, d), jnp.float32),
            pltpu.VMEM((_ZR, d), jnp.float32),
            pltpu.VMEM_SHARED((n, d), jnp.float32),
            pltpu.SemaphoreType.DMA,
        ],
    )
    def scatter_k(edge_hbm, x_hbm, part_hbm, src_v, dst_v, rows_v, zero_v,
                  acc_sh, sem):
        cid = lax.axis_index("c")
        sid = lax.axis_index("s")
        w = cid * _NS + sid

        def _zrow(i, carry):
            for c16 in range(d // 16):
                zero_v[i, pl.ds(c16 * 16, 16)] = jnp.zeros((16,), jnp.float32)
            return carry

        lax.fori_loop(0, _ZR, _zrow, 0)
        for k in range(rpt // _ZR):
            pltpu.sync_copy(zero_v,
                            acc_sh.at[pl.ds(sid * rpt + k * _ZR, _ZR)])

        pltpu.sync_copy(edge_hbm.at[0, w], src_v)
        pltpu.sync_copy(edge_hbm.at[1, w], dst_v)
        plsc.subcore_barrier()

        def _chunk(j, carry):
            pltpu.async_copy(x_hbm.at[src_v.at[j]], rows_v, sem).wait()
            pltpu.sync_copy(rows_v, acc_sh.at[dst_v.at[j]], add=True)
            return carry

        lax.fori_loop(0, nch, _chunk, 0)
        plsc.subcore_barrier()

        pltpu.sync_copy(acc_sh.at[pl.ds(sid * rpt, rpt)],
                        part_hbm.at[cid, pl.ds(sid * rpt, rpt)])

    return scatter_k(eidx, x)


def kernel(x, edge_index, W_neigh, b_neigh, W_self, b_self):
    n, d = x.shape
    d_out = W_neigh.shape[0]
    e = edge_index.shape[1]
    nw = _NC * _NS
    epw = e // nw
    nch = epw // _C
    npad = -(-n // (_NS * _ZR)) * (_NS * _ZR)
    assert e == nw * epw and epw == nch * _C and d % 16 == 0

    eidx = edge_index.reshape(2, nw, nch, _C)
    parts = _neighbor_partials(eidx, x, npad)

    bias = (b_neigh + b_self).reshape(1, d_out)
    bt = 1000

    def combine_body(p_ref, x_ref, wn_ref, ws_ref, b_ref, o_ref):
        neigh = p_ref[0] + p_ref[1]
        o_ref[...] = (
            lax.dot_general(neigh, wn_ref[...], (((1,), (1,)), ((), ())),
                            preferred_element_type=jnp.float32)
            + lax.dot_general(x_ref[...], ws_ref[...], (((1,), (1,)), ((), ())),
                              preferred_element_type=jnp.float32)
            + b_ref[...]
        )

    out = pl.pallas_call(
        combine_body,
        grid=(n // bt,),
        in_specs=[
            pl.BlockSpec((_NC, bt, d), lambda i: (0, i, 0)),
            pl.BlockSpec((bt, d), lambda i: (i, 0)),
            pl.BlockSpec((d_out, d), lambda i: (0, 0)),
            pl.BlockSpec((d_out, d), lambda i: (0, 0)),
            pl.BlockSpec((1, d_out), lambda i: (0, 0)),
        ],
        out_specs=pl.BlockSpec((bt, d_out), lambda i: (i, 0)),
        out_shape=jax.ShapeDtypeStruct((n, d_out), jnp.float32),
    )(parts, x, W_neigh, W_self, bias)
    return out


# speedup vs baseline: 10.1041x; 1.2596x over previous
import functools

import jax
import jax.numpy as jnp
from jax import lax
from jax.experimental import pallas as pl
from jax.experimental.pallas import tpu as pltpu
from jax.experimental.pallas import tpu_sc as plsc

_NC = 2
_NS = 16
_C = 50
_RING = 4
_ZR = 16


def _neighbor_partials(eidx, x, npad):
    n, d = npad, x.shape[1]
    nch = eidx.shape[1]
    rpt = n // _NS
    nquad = nch // _RING

    mesh = plsc.VectorSubcoreMesh(core_axis_name="c", subcore_axis_name="s")

    @functools.partial(
        pl.kernel,
        out_type=jax.ShapeDtypeStruct((_NC, n, d), jnp.float32),
        mesh=mesh,
        scratch_types=[
            [pltpu.VMEM((2, _C), jnp.int32) for _ in range(_RING)],
            [pltpu.VMEM((_C, d), jnp.float32) for _ in range(_RING)],
            pltpu.VMEM((_ZR, d), jnp.float32),
            pltpu.VMEM_SHARED((n, d), jnp.float32),
            [pltpu.SemaphoreType.DMA for _ in range(_RING)],
            [pltpu.SemaphoreType.DMA for _ in range(_RING)],
            [pltpu.SemaphoreType.DMA for _ in range(_RING)],
            pltpu.SemaphoreType.DMA,
        ],
    )
    def scatter_k(edge_hbm, x_hbm, part_hbm, ebufs, rows, zero_v, acc_sh,
                  esem, gsem, ssem, zsem):
        cid = lax.axis_index("c")
        sid = lax.axis_index("s")
        w = cid * _NS + sid

        def _zrow(i, carry):
            for c16 in range(d // 16):
                zero_v[i, pl.ds(c16 * 16, 16)] = jnp.zeros((16,), jnp.float32)
            return carry

        lax.fori_loop(0, _ZR, _zrow, 0)
        nz = rpt // _ZR
        for k in range(nz):
            pltpu.async_copy(zero_v, acc_sh.at[pl.ds(sid * rpt + k * _ZR, _ZR)],
                             zsem)
        for k in range(nz):
            pltpu.make_async_copy(
                zero_v, acc_sh.at[pl.ds(sid * rpt, _ZR)], zsem).wait()
        plsc.subcore_barrier()

        def issue_e(c, b):
            pltpu.async_copy(edge_hbm.at[w, c], ebufs[b], esem[b])

        def wait_e(b):
            pltpu.make_async_copy(edge_hbm.at[w, 0], ebufs[b], esem[b]).wait()

        def issue_g(b):
            pltpu.async_copy(x_hbm.at[ebufs[b].at[0]], rows[b], gsem[b])

        def wait_g(b):
            pltpu.make_async_copy(x_hbm.at[ebufs[b].at[0]], rows[b],
                                  gsem[b]).wait()

        def issue_s(b):
            pltpu.async_copy(rows[b], acc_sh.at[ebufs[b].at[1]], ssem[b],
                             add=True)

        def wait_s(b):
            pltpu.make_async_copy(rows[b], acc_sh.at[ebufs[b].at[1]],
                                  ssem[b]).wait()

        for b in range(_RING):
            issue_e(b, b)
        for b in range(_RING):
            wait_e(b)
            issue_g(b)

        def _quad(q, carry):
            base = q * _RING
            for b in range(_RING):
                wait_g(b)
                issue_s(b)
            for b in range(_RING):
                wait_s(b)
                issue_e(base + _RING + b, b)
            for b in range(_RING):
                wait_e(b)
                issue_g(b)
            return carry

        lax.fori_loop(0, nquad - 1, _quad, 0)
        for b in range(_RING):
            wait_g(b)
            issue_s(b)
        for b in range(_RING):
            wait_s(b)
        plsc.subcore_barrier()

        pltpu.sync_copy(acc_sh.at[pl.ds(sid * rpt, rpt)],
                        part_hbm.at[cid, pl.ds(sid * rpt, rpt)])

    return scatter_k(eidx, x)


def kernel(x, edge_index, W_neigh, b_neigh, W_self, b_self):
    n, d = x.shape
    d_out = W_neigh.shape[0]
    e = edge_index.shape[1]
    nw = _NC * _NS
    epw = e // nw
    nch = epw // _C
    npad = -(-n // (_NS * 128)) * (_NS * 128)
    assert e == nw * epw and epw == nch * _C and nch % _RING == 0
    assert d % 16 == 0 and (npad // _NS) % _ZR == 0

    eidx = edge_index.reshape(2, nw, nch, _C).transpose(1, 2, 0, 3)
    parts = _neighbor_partials(eidx, x, npad)

    bias = (b_neigh + b_self).reshape(1, d_out)
    bt = 1000

    def combine_body(p_ref, x_ref, wn_ref, ws_ref, b_ref, o_ref):
        neigh = p_ref[0] + p_ref[1]
        o_ref[...] = (
            lax.dot_general(neigh, wn_ref[...], (((1,), (1,)), ((), ())),
                            preferred_element_type=jnp.float32)
            + lax.dot_general(x_ref[...], ws_ref[...], (((1,), (1,)), ((), ())),
                              preferred_element_type=jnp.float32)
            + b_ref[...]
        )

    out = pl.pallas_call(
        combine_body,
        grid=(n // bt,),
        in_specs=[
            pl.BlockSpec((_NC, bt, d), lambda i: (0, i, 0)),
            pl.BlockSpec((bt, d), lambda i: (i, 0)),
            pl.BlockSpec((d_out, d), lambda i: (0, 0)),
            pl.BlockSpec((d_out, d), lambda i: (0, 0)),
            pl.BlockSpec((1, d_out), lambda i: (0, 0)),
        ],
        out_specs=pl.BlockSpec((bt, d_out), lambda i: (i, 0)),
        out_shape=jax.ShapeDtypeStruct((n, d_out), jnp.float32),
    )(parts, x, W_neigh, W_self, bias)
    return out
